# baseline (speedup 1.0000x reference)
import functools
import math

import jax
import jax.numpy as jnp
from jax import lax
from jax.experimental import pallas as pl
from jax.experimental.pallas import tpu as pltpu
from jax.experimental.pallas import tpu_sc as plsc

N = 10000
K = 7
TOWERS = 4


RCHUNK = 400
NCHUNKS = N // RCHUNK


_IDXBITS = 14
_IDXMASK = (1 << _IDXBITS) - 1
_IMAX = 0x7FFFFFFF


def _knn_body(p_ref, q_ref, out_ref):
    ci = pl.program_id(0)
    p = p_ref[:]
    q = q_ref[:]
    psq = jnp.sum(p * p, axis=1, keepdims=True)
    qsq = jnp.sum(q * q, axis=0, keepdims=True)
    dist = psq - 2.0 * jnp.dot(p, q, preferred_element_type=jnp.float32) + qsq
    rows = ci * RCHUNK + lax.broadcasted_iota(jnp.int32, (RCHUNK, 1), 0)
    cols = lax.broadcasted_iota(jnp.int32, (RCHUNK, N), 1)
    inf = jnp.float32(jnp.inf)
    dist = jnp.where(cols == rows, inf, dist)
    picks = []
    for _ in range(K):
        m = jnp.min(dist, axis=1, keepdims=True)
        idx = jnp.min(jnp.where(dist == m, cols, N), axis=1, keepdims=True)
        picks.append(idx)
        dist = jnp.where(cols == idx, inf, dist)
    picks.append(jnp.zeros((RCHUNK, 1), jnp.int32))
    out_ref[:] = jnp.concatenate(picks, axis=1)


def _knn(pos_pad, pos_t):
    return pl.pallas_call(
        _knn_body,
        grid=(NCHUNKS,),
        in_specs=[
            pl.BlockSpec((RCHUNK, 8), lambda i: (i, 0)),
            pl.BlockSpec((8, N), lambda i: (0, 0)),
        ],
        out_specs=pl.BlockSpec((RCHUNK, 8), lambda i: (i, 0)),
        out_shape=jax.ShapeDtypeStruct((N, 8), jnp.int32),
    )(pos_pad, pos_t)


MM_BLOCK = 2000


def _mm_body(x_ref, w_ref, o_ref):
    o_ref[:] = jnp.dot(x_ref[:], w_ref[:], preferred_element_type=jnp.float32)


def _matmul(x, w):
    m, kdim = x.shape
    nout = w.shape[1]
    return pl.pallas_call(
        _mm_body,
        grid=(m // MM_BLOCK,),
        in_specs=[
            pl.BlockSpec((MM_BLOCK, kdim), lambda i: (i, 0)),
            pl.BlockSpec((kdim, nout), lambda i: (0, 0)),
        ],
        out_specs=pl.BlockSpec((MM_BLOCK, nout), lambda i: (i, 0)),
        out_shape=jax.ShapeDtypeStruct((m, nout), jnp.float32),
    )(x, w)


_NC = 2
_NS = 16
NW = _NC * _NS
NPAD = 10240
NODES_PER_W = NPAD // NW
G = 8
NG = NODES_PER_W // G
LANES = 16


def _gather_reduce(b, idx_flat, width):
    nch = width // LANES
    mesh = plsc.VectorSubcoreMesh(core_axis_name="c", subcore_axis_name="s")
    out_t = [jax.ShapeDtypeStruct((NPAD, width), jnp.float32) for _ in range(4)]
    row_buf = pltpu.VMEM((G * K, width), jnp.float32)
    out_buf = pltpu.VMEM((G, width), jnp.float32)
    scratch = [
        pltpu.VMEM((NODES_PER_W * K,), jnp.int32),
        row_buf, row_buf,
        out_buf, out_buf, out_buf, out_buf,
        out_buf, out_buf, out_buf, out_buf,
        pltpu.SemaphoreType.DMA, pltpu.SemaphoreType.DMA,
        pltpu.SemaphoreType.DMA, pltpu.SemaphoreType.DMA,
    ]

    @functools.partial(pl.kernel, mesh=mesh, out_type=out_t, scratch_types=scratch)
    def body(b_hbm, idx_hbm, s_hbm, q_hbm, mn_hbm, mx_hbm,
             idx_all, rows0, rows1, s0, q0, n0, x0, s1, q1, n1, x1,
             gsem0, gsem1, ssem0, ssem1):
        cid = lax.axis_index("c")
        sid = lax.axis_index("s")
        wid = sid * _NC + cid
        base0 = wid * NODES_PER_W
        rows = (rows0, rows1)
        outs = ((s0, q0, n0, x0), (s1, q1, n1, x1))
        gsems = (gsem0, gsem1)
        ssems = (ssem0, ssem1)
        out_hbms = (s_hbm, q_hbm, mn_hbm, mx_hbm)

        pltpu.sync_copy(idx_hbm.at[pl.ds(base0 * K, NODES_PER_W * K)], idx_all)
        for bi in range(2):
            pltpu.async_copy(
                b_hbm.at[idx_all.at[pl.ds(bi * G * K, G * K)]], rows[bi], gsems[bi])

        def round2(gb, carry):
            for bi in range(2):
                g = gb * 2 + bi
                node0 = base0 + g * G
                rv = rows[bi]
                pltpu.make_async_copy(
                    b_hbm.at[pl.ds(0, G * K)], rv, gsems[bi]).wait()
                @pl.when(gb > 0)
                def _():
                    for oi in range(4):
                        pltpu.make_async_copy(
                            outs[bi][oi], out_hbms[oi].at[pl.ds(0, G)],
                            ssems[bi]).wait()

                def per_quad(i4, carry2):
                    i0 = i4 * 4
                    for u in range(4):
                        for ch in range(nch):
                            sl = pl.ds(ch * LANES, LANES)
                            v = rv[(i0 + u) * K, sl]
                            acc_s = v
                            acc_q = v * v
                            acc_n = v
                            acc_x = v
                            for kk in range(1, K):
                                v = rv[(i0 + u) * K + kk, sl]
                                acc_s = acc_s + v
                                acc_q = acc_q + v * v
                                acc_n = jnp.minimum(acc_n, v)
                                acc_x = jnp.maximum(acc_x, v)
                            outs[bi][0][i0 + u, sl] = acc_s
                            outs[bi][1][i0 + u, sl] = acc_q
                            outs[bi][2][i0 + u, sl] = acc_n
                            outs[bi][3][i0 + u, sl] = acc_x
                    return carry2

                lax.fori_loop(0, G // 4, per_quad, 0)
                @pl.when(g + 2 < NG)
                def _():
                    pltpu.async_copy(
                        b_hbm.at[idx_all.at[pl.ds((g + 2) * G * K, G * K)]],
                        rv, gsems[bi])
                for oi in range(4):
                    pltpu.async_copy(
                        outs[bi][oi], out_hbms[oi].at[pl.ds(node0, G)], ssems[bi])
            return carry

        lax.fori_loop(0, NG // 2, round2, 0)
        for bi in range(2):
            for oi in range(4):
                pltpu.make_async_copy(
                    outs[bi][oi], out_hbms[oi].at[pl.ds(0, G)], ssems[bi]).wait()

    return body(b, idx_flat)


EPI_BLOCK = 2000


def _epi_body(x_ref, s_ref, q_ref, n_ref, m_ref,
              gx_ref, cm_ref, cn_ref, cx_ref, cs_ref, c0_ref, o_ref):
    s = s_ref[:]
    sm = s * (1.0 / 7.0)
    var = q_ref[:] * (1.0 / 7.0) - sm * sm
    std = jnp.sqrt(jnp.maximum(var, 0.0) + 1e-5)
    acc = jnp.dot(x_ref[:], gx_ref[:], preferred_element_type=jnp.float32)
    acc = acc + jnp.dot(sm, cm_ref[:], preferred_element_type=jnp.float32)
    acc = acc + jnp.dot(n_ref[:], cn_ref[:], preferred_element_type=jnp.float32)
    acc = acc + jnp.dot(m_ref[:], cx_ref[:], preferred_element_type=jnp.float32)
    acc = acc + jnp.dot(std, cs_ref[:], preferred_element_type=jnp.float32)
    o_ref[:] = acc + c0_ref[:]


def _epilogue(x, s, q, mn, mx, gx, cm, cn, cx, cs, c0):
    m, din = x.shape
    w4 = s.shape[1]
    h = gx.shape[1]
    bs = lambda shape: shape
    return pl.pallas_call(
        _epi_body,
        grid=(m // EPI_BLOCK,),
        in_specs=[
            pl.BlockSpec((EPI_BLOCK, din), lambda i: (i, 0)),
            pl.BlockSpec((EPI_BLOCK, w4), lambda i: (i, 0)),
            pl.BlockSpec((EPI_BLOCK, w4), lambda i: (i, 0)),
            pl.BlockSpec((EPI_BLOCK, w4), lambda i: (i, 0)),
            pl.BlockSpec((EPI_BLOCK, w4), lambda i: (i, 0)),
            pl.BlockSpec((din, h), lambda i: (0, 0)),
            pl.BlockSpec((w4, h), lambda i: (0, 0)),
            pl.BlockSpec((w4, h), lambda i: (0, 0)),
            pl.BlockSpec((w4, h), lambda i: (0, 0)),
            pl.BlockSpec((w4, h), lambda i: (0, 0)),
            pl.BlockSpec((1, h), lambda i: (0, 0)),
        ],
        out_specs=pl.BlockSpec((EPI_BLOCK, h), lambda i: (i, 0)),
        out_shape=jax.ShapeDtypeStruct((m, h), jnp.float32),
    )(x, s, q, mn, mx, gx, cm, cn, cx, cs, c0)


def _bn_body(h_ref, g_ref, b_ref, o_ref):
    h = h_ref[:]
    mu = jnp.mean(h, axis=0, keepdims=True)
    var = jnp.mean((h - mu) ** 2, axis=0, keepdims=True)
    o_ref[:] = jnp.maximum(g_ref[:] * (h - mu) / jnp.sqrt(var + 1e-5) + b_ref[:], 0.0)


def _bn_relu(h, gamma, beta):
    m, c = h.shape
    return pl.pallas_call(
        _bn_body,
        in_specs=[
            pl.BlockSpec((m, c), lambda: (0, 0)),
            pl.BlockSpec((1, c), lambda: (0, 0)),
            pl.BlockSpec((1, c), lambda: (0, 0)),
        ],
        out_specs=pl.BlockSpec((m, c), lambda: (0, 0)),
        out_shape=jax.ShapeDtypeStruct((m, c), jnp.float32),
    )(h, gamma.reshape(1, c), beta.reshape(1, c))


def _bn_pool_body(h_ref, g_ref, b_ref, o_ref):
    h = h_ref[:]
    mu = jnp.mean(h, axis=0, keepdims=True)
    var = jnp.mean((h - mu) ** 2, axis=0, keepdims=True)
    hn = jnp.maximum(g_ref[:] * (h - mu) / jnp.sqrt(var + 1e-5) + b_ref[:], 0.0)
    o_ref[:] = jnp.mean(hn, axis=0, keepdims=True)


def _bn_relu_pool(h, gamma, beta):
    m, c = h.shape
    return pl.pallas_call(
        _bn_pool_body,
        in_specs=[
            pl.BlockSpec((m, c), lambda: (0, 0)),
            pl.BlockSpec((1, c), lambda: (0, 0)),
            pl.BlockSpec((1, c), lambda: (0, 0)),
        ],
        out_specs=pl.BlockSpec((1, c), lambda: (0, 0)),
        out_shape=jax.ShapeDtypeStruct((1, c), jnp.float32),
    )(h, gamma.reshape(1, c), beta.reshape(1, c))


def _fold(pre_W, pre_b, post_W, post_b, lin_W, lin_b):
    d = pre_W.shape[2]
    dp = post_W.shape[2]
    h = lin_W.shape[1]
    wd = pre_W[:, :d, :]
    ws = pre_W[:, d:, :]
    wsrc = jnp.concatenate([ws[t] for t in range(TOWERS)], axis=1)
    lt = lin_W.reshape(TOWERS, dp, h)
    p_x = post_W[:, 0:d]
    p_m = post_W[:, d:2 * d] + post_W[:, 5 * d:6 * d] + post_W[:, 9 * d:10 * d]
    p_n = post_W[:, 2 * d:3 * d] + post_W[:, 6 * d:7 * d] + post_W[:, 10 * d:11 * d]
    p_X = post_W[:, 3 * d:4 * d] + post_W[:, 7 * d:8 * d] + post_W[:, 11 * d:12 * d]
    p_s = post_W[:, 4 * d:5 * d] + post_W[:, 8 * d:9 * d] + post_W[:, 12 * d:13 * d]
    qx = jnp.einsum('tdp,tph->tdh', p_x, lt)
    qm = jnp.einsum('tdp,tph->tdh', p_m, lt)
    qn = jnp.einsum('tdp,tph->tdh', p_n, lt)
    qX = jnp.einsum('tdp,tph->tdh', p_X, lt)
    qs = jnp.einsum('tdp,tph->tdh', p_s, lt)
    qa = qm + qn + qX
    gx = qx.sum(0) + jnp.einsum('tde,teh->dh', wd, qa)
    c0 = (lin_b + jnp.einsum('tp,tph->h', post_b, lt)
          + jnp.einsum('td,tdh->h', pre_b, qa))
    cat = lambda q: jnp.concatenate([q[t] for t in range(TOWERS)], axis=0)
    cm = cat(qm)
    cn = cat(qn)
    cx = cat(qX)
    cs = cat(qs)
    return wsrc, gx, cm, cn, cx, cs, c0.reshape(1, h)


def _layer(xin, idx_flat, fold):
    wsrc, gx, cm, cn, cx, cs, c0 = fold
    b = _matmul(xin, wsrc)
    s, q, mn, mx = _gather_reduce(b, idx_flat, wsrc.shape[1])
    return _epilogue(xin, s[:N], q[:N], mn[:N], mx[:N], gx, cm, cn, cx, cs, c0)


def kernel(x, pos, batch, pre_W1, pre_b1, post_W1, post_b1, lin_W1, lin_b1,
           bn1_g, bn1_b, pre_W2, pre_b2, post_W2, post_b2, lin_W2, lin_b2,
           bn2_g, bn2_b):
    pos_pad = jnp.pad(pos, ((0, 0), (0, 5)))
    src8 = _knn(pos_pad, pos_pad.T)
    idx_flat = jnp.pad(src8[:, :K].reshape(-1), (0, NPAD * K - N * K))

    f1 = _fold(pre_W1, pre_b1, post_W1, post_b1, lin_W1, lin_b1)
    h = _layer(x, idx_flat, f1)
    h = _bn_relu(h, bn1_g, bn1_b)

    f2 = _fold(pre_W2, pre_b2, post_W2, post_b2, lin_W2, lin_b2)
    h2 = _layer(h, idx_flat, f2)
    return _bn_relu_pool(h2, bn2_g, bn2_b)


# speedup vs baseline: 1.4888x; 1.4888x over previous
import functools
import math

import jax
import jax.numpy as jnp
from jax import lax
from jax.experimental import pallas as pl
from jax.experimental.pallas import tpu as pltpu
from jax.experimental.pallas import tpu_sc as plsc

N = 10000
K = 7
TOWERS = 4


RCHUNK = 256


def _knn_body(r0, p_ref, q_ref, out_ref):
    ci = pl.program_id(0)
    p = p_ref[:]
    q = q_ref[:]
    psq = jnp.sum(p * p, axis=1, keepdims=True)
    qsq = jnp.sum(q * q, axis=0, keepdims=True)
    dist = psq - 2.0 * jnp.dot(p, q, preferred_element_type=jnp.float32) + qsq
    rows = r0 + ci * RCHUNK + lax.broadcasted_iota(jnp.int32, (RCHUNK, 1), 0)
    cols = lax.broadcasted_iota(jnp.int32, (RCHUNK, N), 1)
    inf = jnp.float32(jnp.inf)
    dist = jnp.where(cols == rows, inf, dist)
    picks = []
    for _ in range(K):
        m = jnp.min(dist, axis=1, keepdims=True)
        idx = jnp.min(jnp.where(dist == m, cols, N), axis=1, keepdims=True)
        picks.append(idx)
        dist = jnp.where(cols == idx, inf, dist)
    picks.append(jnp.zeros((RCHUNK, 1), jnp.int32))
    out_ref[:] = jnp.concatenate(picks, axis=1)


def _knn_range(p_rows, pos_t, r0, nrows):
    return pl.pallas_call(
        functools.partial(_knn_body, r0),
        grid=(nrows // RCHUNK,),
        in_specs=[
            pl.BlockSpec((RCHUNK, 8), lambda i: (i, 0)),
            pl.BlockSpec((8, N), lambda i: (0, 0)),
        ],
        out_specs=pl.BlockSpec((RCHUNK, 8), lambda i: (i, 0)),
        out_shape=jax.ShapeDtypeStruct((nrows, 8), jnp.int32),
    )(p_rows, pos_t)


MM_BLOCK = 2048


def _mm_body(x_ref, w_ref, o_ref):
    o_ref[:] = jnp.dot(x_ref[:], w_ref[:], preferred_element_type=jnp.float32)


def _matmul(x, w):
    m, kdim = x.shape
    nout = w.shape[1]
    return pl.pallas_call(
        _mm_body,
        grid=(m // MM_BLOCK,),
        in_specs=[
            pl.BlockSpec((MM_BLOCK, kdim), lambda i: (i, 0)),
            pl.BlockSpec((kdim, nout), lambda i: (0, 0)),
        ],
        out_specs=pl.BlockSpec((MM_BLOCK, nout), lambda i: (i, 0)),
        out_shape=jax.ShapeDtypeStruct((m, nout), jnp.float32),
    )(x, w)


_NC = 2
_NS = 16
NW = _NC * _NS
NPAD = 10240
RA = 6144
RB = NPAD - RA
G = 8
LANES = 16


def _gather_reduce(b, idx_flat, width, n_nodes):
    nch = width // LANES
    nodes_per_w = n_nodes // NW
    ng = nodes_per_w // G
    mesh = plsc.VectorSubcoreMesh(core_axis_name="c", subcore_axis_name="s")
    out_t = [jax.ShapeDtypeStruct((n_nodes, width), jnp.float32) for _ in range(4)]
    row_buf = pltpu.VMEM((G * K, width), jnp.float32)
    out_buf = pltpu.VMEM((G, width), jnp.float32)
    scratch = [
        pltpu.VMEM((nodes_per_w * K,), jnp.int32),
        row_buf, row_buf,
        out_buf, out_buf, out_buf, out_buf,
        out_buf, out_buf, out_buf, out_buf,
        pltpu.SemaphoreType.DMA, pltpu.SemaphoreType.DMA,
        pltpu.SemaphoreType.DMA, pltpu.SemaphoreType.DMA,
    ]

    @functools.partial(pl.kernel, mesh=mesh, out_type=out_t, scratch_types=scratch)
    def body(b_hbm, idx_hbm, s_hbm, q_hbm, mn_hbm, mx_hbm,
             idx_all, rows0, rows1, s0, q0, n0, x0, s1, q1, n1, x1,
             gsem0, gsem1, ssem0, ssem1):
        cid = lax.axis_index("c")
        sid = lax.axis_index("s")
        wid = sid * _NC + cid
        base0 = wid * nodes_per_w
        rows = (rows0, rows1)
        outs = ((s0, q0, n0, x0), (s1, q1, n1, x1))
        gsems = (gsem0, gsem1)
        ssems = (ssem0, ssem1)
        out_hbms = (s_hbm, q_hbm, mn_hbm, mx_hbm)

        pltpu.sync_copy(idx_hbm.at[pl.ds(base0 * K, nodes_per_w * K)], idx_all)
        for bi in range(2):
            pltpu.async_copy(
                b_hbm.at[idx_all.at[pl.ds(bi * G * K, G * K)]], rows[bi], gsems[bi])

        def round2(gb, carry):
            for bi in range(2):
                g = gb * 2 + bi
                node0 = base0 + g * G
                rv = rows[bi]
                pltpu.make_async_copy(
                    b_hbm.at[pl.ds(0, G * K)], rv, gsems[bi]).wait()
                @pl.when(gb > 0)
                def _():
                    for oi in range(4):
                        pltpu.make_async_copy(
                            outs[bi][oi], out_hbms[oi].at[pl.ds(0, G)],
                            ssems[bi]).wait()

                def per_node(i, carry2):
                    for ch in range(nch):
                        sl = pl.ds(ch * LANES, LANES)
                        v = rv[i * K, sl]
                        acc_s = v
                        acc_q = v * v
                        acc_n = v
                        acc_x = v
                        for kk in range(1, K):
                            v = rv[i * K + kk, sl]
                            acc_s = acc_s + v
                            acc_q = acc_q + v * v
                            acc_n = jnp.minimum(acc_n, v)
                            acc_x = jnp.maximum(acc_x, v)
                        outs[bi][0][i, sl] = acc_s
                        outs[bi][1][i, sl] = acc_q
                        outs[bi][2][i, sl] = acc_n
                        outs[bi][3][i, sl] = acc_x
                    return carry2

                lax.fori_loop(0, G, per_node, 0)
                @pl.when(g + 2 < ng)
                def _():
                    pltpu.async_copy(
                        b_hbm.at[idx_all.at[pl.ds((g + 2) * G * K, G * K)]],
                        rv, gsems[bi])
                for oi in range(4):
                    pltpu.async_copy(
                        outs[bi][oi], out_hbms[oi].at[pl.ds(node0, G)], ssems[bi])
            return carry

        lax.fori_loop(0, ng // 2, round2, 0)
        for bi in range(2):
            for oi in range(4):
                pltpu.make_async_copy(
                    outs[bi][oi], out_hbms[oi].at[pl.ds(0, G)], ssems[bi]).wait()

    return body(b, idx_flat)


EPI_BLOCK = 2048


def _epi_body(x_ref, s_ref, q_ref, n_ref, m_ref,
              gx_ref, cm_ref, cn_ref, cx_ref, cs_ref, c0_ref, o_ref):
    s = s_ref[:]
    sm = s * (1.0 / 7.0)
    var = q_ref[:] * (1.0 / 7.0) - sm * sm
    std = jnp.sqrt(jnp.maximum(var, 0.0) + 1e-5)
    acc = jnp.dot(x_ref[:], gx_ref[:], preferred_element_type=jnp.float32)
    acc = acc + jnp.dot(sm, cm_ref[:], preferred_element_type=jnp.float32)
    acc = acc + jnp.dot(n_ref[:], cn_ref[:], preferred_element_type=jnp.float32)
    acc = acc + jnp.dot(m_ref[:], cx_ref[:], preferred_element_type=jnp.float32)
    acc = acc + jnp.dot(std, cs_ref[:], preferred_element_type=jnp.float32)
    o_ref[:] = acc + c0_ref[:]


def _epilogue(x, s, q, mn, mx, gx, cm, cn, cx, cs, c0):
    m, din = x.shape
    w4 = s.shape[1]
    h = gx.shape[1]
    bs = lambda shape: shape
    return pl.pallas_call(
        _epi_body,
        grid=(m // EPI_BLOCK,),
        in_specs=[
            pl.BlockSpec((EPI_BLOCK, din), lambda i: (i, 0)),
            pl.BlockSpec((EPI_BLOCK, w4), lambda i: (i, 0)),
            pl.BlockSpec((EPI_BLOCK, w4), lambda i: (i, 0)),
            pl.BlockSpec((EPI_BLOCK, w4), lambda i: (i, 0)),
            pl.BlockSpec((EPI_BLOCK, w4), lambda i: (i, 0)),
            pl.BlockSpec((din, h), lambda i: (0, 0)),
            pl.BlockSpec((w4, h), lambda i: (0, 0)),
            pl.BlockSpec((w4, h), lambda i: (0, 0)),
            pl.BlockSpec((w4, h), lambda i: (0, 0)),
            pl.BlockSpec((w4, h), lambda i: (0, 0)),
            pl.BlockSpec((1, h), lambda i: (0, 0)),
        ],
        out_specs=pl.BlockSpec((EPI_BLOCK, h), lambda i: (i, 0)),
        out_shape=jax.ShapeDtypeStruct((m, h), jnp.float32),
    )(x, s, q, mn, mx, gx, cm, cn, cx, cs, c0)


def _bn_body(h_ref, g_ref, b_ref, o_ref):
    h = h_ref[:]
    mu = jnp.mean(h, axis=0, keepdims=True)
    var = jnp.mean((h - mu) ** 2, axis=0, keepdims=True)
    o_ref[:] = jnp.maximum(g_ref[:] * (h - mu) / jnp.sqrt(var + 1e-5) + b_ref[:], 0.0)


def _bn_relu(h, gamma, beta):
    m, c = h.shape
    return pl.pallas_call(
        _bn_body,
        in_specs=[
            pl.BlockSpec((m, c), lambda: (0, 0)),
            pl.BlockSpec((1, c), lambda: (0, 0)),
            pl.BlockSpec((1, c), lambda: (0, 0)),
        ],
        out_specs=pl.BlockSpec((m, c), lambda: (0, 0)),
        out_shape=jax.ShapeDtypeStruct((m, c), jnp.float32),
    )(h, gamma.reshape(1, c), beta.reshape(1, c))


def _bn_pool_body(h_ref, g_ref, b_ref, o_ref):
    h = h_ref[:]
    mu = jnp.mean(h, axis=0, keepdims=True)
    var = jnp.mean((h - mu) ** 2, axis=0, keepdims=True)
    hn = jnp.maximum(g_ref[:] * (h - mu) / jnp.sqrt(var + 1e-5) + b_ref[:], 0.0)
    o_ref[:] = jnp.mean(hn, axis=0, keepdims=True)


def _bn_relu_pool(h, gamma, beta):
    m, c = h.shape
    return pl.pallas_call(
        _bn_pool_body,
        in_specs=[
            pl.BlockSpec((m, c), lambda: (0, 0)),
            pl.BlockSpec((1, c), lambda: (0, 0)),
            pl.BlockSpec((1, c), lambda: (0, 0)),
        ],
        out_specs=pl.BlockSpec((1, c), lambda: (0, 0)),
        out_shape=jax.ShapeDtypeStruct((1, c), jnp.float32),
    )(h, gamma.reshape(1, c), beta.reshape(1, c))


def _fold(pre_W, pre_b, post_W, post_b, lin_W, lin_b):
    d = pre_W.shape[2]
    dp = post_W.shape[2]
    h = lin_W.shape[1]
    wd = pre_W[:, :d, :]
    ws = pre_W[:, d:, :]
    wsrc = jnp.concatenate([ws[t] for t in range(TOWERS)], axis=1)
    lt = lin_W.reshape(TOWERS, dp, h)
    p_x = post_W[:, 0:d]
    p_m = post_W[:, d:2 * d] + post_W[:, 5 * d:6 * d] + post_W[:, 9 * d:10 * d]
    p_n = post_W[:, 2 * d:3 * d] + post_W[:, 6 * d:7 * d] + post_W[:, 10 * d:11 * d]
    p_X = post_W[:, 3 * d:4 * d] + post_W[:, 7 * d:8 * d] + post_W[:, 11 * d:12 * d]
    p_s = post_W[:, 4 * d:5 * d] + post_W[:, 8 * d:9 * d] + post_W[:, 12 * d:13 * d]
    qx = jnp.einsum('tdp,tph->tdh', p_x, lt)
    qm = jnp.einsum('tdp,tph->tdh', p_m, lt)
    qn = jnp.einsum('tdp,tph->tdh', p_n, lt)
    qX = jnp.einsum('tdp,tph->tdh', p_X, lt)
    qs = jnp.einsum('tdp,tph->tdh', p_s, lt)
    qa = qm + qn + qX
    gx = qx.sum(0) + jnp.einsum('tde,teh->dh', wd, qa)
    c0 = (lin_b + jnp.einsum('tp,tph->h', post_b, lt)
          + jnp.einsum('td,tdh->h', pre_b, qa))
    cat = lambda q: jnp.concatenate([q[t] for t in range(TOWERS)], axis=0)
    cm = cat(qm)
    cn = cat(qn)
    cx = cat(qX)
    cs = cat(qs)
    return wsrc, gx, cm, cn, cx, cs, c0.reshape(1, h)


def _layer(xp, idx_a, idx_b, fold):
    wsrc, gx, cm, cn, cx, cs, c0 = fold
    b = _matmul(xp, wsrc)
    sa, qa, na, xa = _gather_reduce(b, idx_a, wsrc.shape[1], RA)
    sb, qb, nb, xb = _gather_reduce(b, idx_b, wsrc.shape[1], RB)
    ha = _epilogue(xp[:RA], sa, qa, na, xa, gx, cm, cn, cx, cs, c0)
    hb = _epilogue(xp[RA:], sb, qb, nb, xb, gx, cm, cn, cx, cs, c0)
    return jnp.concatenate([ha, hb], axis=0)[:N]


def kernel(x, pos, batch, pre_W1, pre_b1, post_W1, post_b1, lin_W1, lin_b1,
           bn1_g, bn1_b, pre_W2, pre_b2, post_W2, post_b2, lin_W2, lin_b2,
           bn2_g, bn2_b):
    pos_pad = jnp.pad(pos, ((0, NPAD - N), (0, 5)))
    pos_t = pos_pad[:N].T
    src_a = _knn_range(pos_pad[:RA], pos_t, 0, RA)
    src_b = _knn_range(pos_pad[RA:], pos_t, RA, RB)
    idx_a = src_a[:, :K].reshape(-1)
    idx_b = src_b[:, :K].reshape(-1)
    ar = jnp.arange(RB * K, dtype=jnp.int32)
    idx_b = jnp.where(ar < (RB - (NPAD - N)) * K, idx_b, ar % jnp.int32(N))

    xp = jnp.pad(x, ((0, NPAD - N), (0, 0)))
    f1 = _fold(pre_W1, pre_b1, post_W1, post_b1, lin_W1, lin_b1)
    h = _layer(xp, idx_a, idx_b, f1)
    h = _bn_relu(h, bn1_g, bn1_b)

    f2 = _fold(pre_W2, pre_b2, post_W2, post_b2, lin_W2, lin_b2)
    hp = jnp.pad(h, ((0, NPAD - N), (0, 0)))
    h2 = _layer(hp, idx_a, idx_b, f2)
    return _bn_relu_pool(h2, bn2_g, bn2_b)
